# baseline (speedup 1.0000x reference)
import jax
import jax.numpy as jnp
from jax.experimental import pallas as pl
from jax.experimental.pallas import tpu as pltpu
from jax.experimental.pallas import tpu_sc as plsc

N = 4096
D = 128
K = 2048

IB = 512
AB = 1024
MB = 512
JB = 512
NMB = N // MB
NJB = K // JB
WIN = 128

F8 = jnp.float8_e4m3fn


def _rank_kernel(v_row_ref, v_col_ref, h_ref, g_ref, rank_i_ref, hv_ref,
                 gf8_ref):
    pid = pl.program_id(0)
    v_row = v_row_ref[...]
    v_col = v_col_ref[...]
    gt = v_row > v_col
    eq = v_row == v_col
    jj = jax.lax.broadcasted_iota(jnp.int32, (IB, N), 1)
    ii = jax.lax.broadcasted_iota(jnp.int32, (IB, N), 0) + pid * IB
    cnt = gt.astype(jnp.float32) + (eq & (jj < ii)).astype(jnp.float32)
    rank = jnp.sum(cnt, axis=1, keepdims=True)
    rank_i_ref[...] = rank.astype(jnp.int32)
    hv_ref[...] = h_ref[...] * v_col
    gf8_ref[...] = (g_ref[...] != 0).astype(F8)


def _build_m_kernel(rank_row_ref, m_ref):
    pid = pl.program_id(0)
    rank_row = rank_row_ref[...]
    rr = jax.lax.broadcasted_iota(jnp.int32, (IB, N), 0) + pid * IB
    m_ref[...] = (rank_row == rr).astype(F8)


def _build_a_kernel(m_ref, gf8_ref, a_ref):
    a_ref[...] = jnp.dot(m_ref[...], gf8_ref[...],
                         preferred_element_type=jnp.float32).astype(F8)


def _twohop_kernel(a_ref, gf8_ref, rank_row_ref, rank_col_ref, out_ref,
                   idx_ref, db_s, deg_s):
    s = pl.program_id(0)

    @pl.when(s < NMB)
    def _():
        d = jnp.dot(a_ref[...], gf8_ref[...],
                    preferred_element_type=jnp.float32)
        db_s[:, pl.ds(s * MB, MB)] = (d != 0.0).astype(F8)

    @pl.when(s == NMB - 1)
    def _():
        sel_row = (rank_row_ref[...] < K).astype(F8)
        deg_s[...] = jax.lax.dot_general(
            sel_row, db_s[...], (((1,), (1,)), ((), ())),
            preferred_element_type=jnp.float32)

    @pl.when(s >= NMB)
    def _():
        jc = s - NMB
        rank_col = rank_col_ref[...]
        jj = jax.lax.broadcasted_iota(jnp.int32, (N, JB), 1) + jc * JB
        pb = rank_col == jj
        c = jnp.dot(db_s[...], pb.astype(F8), preferred_element_type=jnp.float32)
        out_ref[...] = c / deg_s[0:1, pl.ds(jc * JB, JB)]
        mi = jax.lax.broadcasted_iota(jnp.int32, (N, JB), 0)
        idx_ref[...] = jnp.sum(jnp.where(pb, mi, 0), axis=0, keepdims=True)


def _sc_scatter_rows(hv, rank_i32):
    rank2 = rank_i32.reshape(1, N)
    mesh = plsc.VectorSubcoreMesh(core_axis_name="c", subcore_axis_name="s")

    @pl.kernel(out_type=jax.ShapeDtypeStruct((N, D), jnp.float32), mesh=mesh)
    def k(hv_hbm, r_hbm, o_hbm):
        def body(hv_vmem, r_vmem):
            pltpu.sync_copy(hv_vmem, o_hbm.at[r_vmem.at[0]])

        pltpu.emit_pipeline(
            body,
            grid=(N // WIN,),
            in_specs=[pl.BlockSpec((WIN, D), lambda i: (i, 0)),
                      pl.BlockSpec((1, WIN), lambda i: (0, i))],
            out_specs=[],
            core_axis_name=("c", "s"),
            dimension_semantics=(pltpu.PARALLEL,),
        )(hv_hbm, r_hbm)

    return k(hv, rank2)


def kernel(g, h, W, b):
    weights = (h @ W + b).squeeze(-1)
    v = jax.nn.sigmoid(weights)
    v_row = v.reshape(1, N)
    v_col = v.reshape(N, 1)

    rank_i, hv, gf8 = pl.pallas_call(
        _rank_kernel,
        grid=(N // IB,),
        in_specs=[
            pl.BlockSpec((1, N), lambda i: (0, 0)),
            pl.BlockSpec((IB, 1), lambda i: (i, 0)),
            pl.BlockSpec((IB, D), lambda i: (i, 0)),
            pl.BlockSpec((IB, N), lambda i: (i, 0)),
        ],
        out_specs=[
            pl.BlockSpec((IB, 1), lambda i: (i, 0)),
            pl.BlockSpec((IB, D), lambda i: (i, 0)),
            pl.BlockSpec((IB, N), lambda i: (i, 0)),
        ],
        out_shape=[
            jax.ShapeDtypeStruct((N, 1), jnp.int32),
            jax.ShapeDtypeStruct((N, D), jnp.float32),
            jax.ShapeDtypeStruct((N, N), F8),
        ],
    )(v_row, v_col, h, g)

    scat = _sc_scatter_rows(hv, rank_i)

    rank_row = rank_i.reshape(1, N)

    m = pl.pallas_call(
        _build_m_kernel,
        grid=(K // IB,),
        in_specs=[pl.BlockSpec((1, N), lambda i: (0, 0))],
        out_specs=pl.BlockSpec((IB, N), lambda i: (i, 0)),
        out_shape=jax.ShapeDtypeStruct((K, N), F8),
    )(rank_row)

    a = pl.pallas_call(
        _build_a_kernel,
        grid=(N // AB,),
        in_specs=[
            pl.BlockSpec((K, N), lambda i: (0, 0)),
            pl.BlockSpec((N, AB), lambda i: (0, i)),
        ],
        out_specs=pl.BlockSpec((K, AB), lambda i: (0, i)),
        out_shape=jax.ShapeDtypeStruct((K, N), F8),
    )(m, gf8)

    g_out, idx_row = pl.pallas_call(
        _twohop_kernel,
        grid=(NMB + NJB,),
        in_specs=[
            pl.BlockSpec((K, N), lambda s: (0, 0)),
            pl.BlockSpec((N, MB), lambda s: (0, jnp.minimum(s, NMB - 1))),
            pl.BlockSpec((1, N), lambda s: (0, 0)),
            pl.BlockSpec((N, 1), lambda s: (0, 0)),
        ],
        out_specs=[
            pl.BlockSpec((K, JB), lambda s: (0, jnp.maximum(s - NMB, 0))),
            pl.BlockSpec((1, JB), lambda s: (0, jnp.maximum(s - NMB, 0))),
        ],
        out_shape=[
            jax.ShapeDtypeStruct((K, K), jnp.float32),
            jax.ShapeDtypeStruct((1, K), jnp.int32),
        ],
        scratch_shapes=[pltpu.VMEM((K, N), F8),
                        pltpu.VMEM((1, K), jnp.float32)],
    )(a, gf8, rank_row, rank_i)

    return (g_out, scat[:K], idx_row.reshape(K))


# speedup vs baseline: 1.0222x; 1.0222x over previous
import jax
import jax.numpy as jnp
from jax.experimental import pallas as pl
from jax.experimental.pallas import tpu as pltpu
from jax.experimental.pallas import tpu_sc as plsc

N = 4096
D = 128
K = 2048

IB = 512
AB = 1024
MB = 1024
JB = 512
NMB = N // MB
NJB = K // JB
WIN = 128

F8 = jnp.float8_e4m3fn


def _rank_kernel(v_row_ref, v_col_ref, h_ref, g_ref, rank_i_ref, hv_ref,
                 gf8_ref):
    pid = pl.program_id(0)
    v_row = v_row_ref[...]
    v_col = v_col_ref[...]
    gt = v_row > v_col
    eq = v_row == v_col
    jj = jax.lax.broadcasted_iota(jnp.int32, (IB, N), 1)
    ii = jax.lax.broadcasted_iota(jnp.int32, (IB, N), 0) + pid * IB
    cnt = gt.astype(jnp.float32) + (eq & (jj < ii)).astype(jnp.float32)
    rank = jnp.sum(cnt, axis=1, keepdims=True)
    rank_i_ref[...] = rank.astype(jnp.int32)
    hv_ref[...] = h_ref[...] * v_col
    gf8_ref[...] = (g_ref[...] != 0).astype(F8)


def _build_a_kernel(rank_row_ref, gf8_ref, a_ref, m_scratch):
    pid = pl.program_id(0)

    @pl.when(pid == 0)
    def _():
        rank_row = rank_row_ref[...]
        for rc in range(K // IB):
            rr = jax.lax.broadcasted_iota(jnp.int32, (IB, N), 0) + (rc * IB)
            m_scratch[rc * IB:(rc + 1) * IB, :] = (rank_row == rr).astype(F8)

    a_ref[...] = jnp.dot(m_scratch[...], gf8_ref[...],
                         preferred_element_type=jnp.float32).astype(F8)


def _twohop_kernel(a_ref, gf8_ref, rank_row_ref, rank_col_ref, out_ref,
                   idx_ref, db_s, deg_s):
    s = pl.program_id(0)

    @pl.when(s < NMB)
    def _():
        d = jnp.dot(a_ref[...], gf8_ref[...],
                    preferred_element_type=jnp.float32)
        db_s[:, pl.ds(s * MB, MB)] = (d != 0.0).astype(F8)

    @pl.when(s == NMB - 1)
    def _():
        sel_row = (rank_row_ref[...] < K).astype(F8)
        deg_s[...] = jax.lax.dot_general(
            sel_row, db_s[...], (((1,), (1,)), ((), ())),
            preferred_element_type=jnp.float32)

    @pl.when(s >= NMB)
    def _():
        jc = s - NMB
        rank_col = rank_col_ref[...]
        jj = jax.lax.broadcasted_iota(jnp.int32, (N, JB), 1) + jc * JB
        pb = rank_col == jj
        c = jnp.dot(db_s[...], pb.astype(F8), preferred_element_type=jnp.float32)
        out_ref[...] = c / deg_s[0:1, pl.ds(jc * JB, JB)]
        mi = jax.lax.broadcasted_iota(jnp.int32, (N, JB), 0)
        idx_ref[...] = jnp.sum(jnp.where(pb, mi, 0), axis=0, keepdims=True)


def _sc_scatter_rows(hv, rank_i32):
    rank2 = rank_i32.reshape(1, N)
    mesh = plsc.VectorSubcoreMesh(core_axis_name="c", subcore_axis_name="s")

    @pl.kernel(out_type=jax.ShapeDtypeStruct((N, D), jnp.float32), mesh=mesh)
    def k(hv_hbm, r_hbm, o_hbm):
        def body(hv_vmem, r_vmem):
            pltpu.sync_copy(hv_vmem, o_hbm.at[r_vmem.at[0]])

        pltpu.emit_pipeline(
            body,
            grid=(N // WIN,),
            in_specs=[pl.BlockSpec((WIN, D), lambda i: (i, 0)),
                      pl.BlockSpec((1, WIN), lambda i: (0, i))],
            out_specs=[],
            core_axis_name=("c", "s"),
            dimension_semantics=(pltpu.PARALLEL,),
        )(hv_hbm, r_hbm)

    return k(hv, rank2)


def kernel(g, h, W, b):
    weights = (h @ W + b).squeeze(-1)
    v = jax.nn.sigmoid(weights)
    v_row = v.reshape(1, N)
    v_col = v.reshape(N, 1)

    rank_i, hv, gf8 = pl.pallas_call(
        _rank_kernel,
        grid=(N // IB,),
        in_specs=[
            pl.BlockSpec((1, N), lambda i: (0, 0)),
            pl.BlockSpec((IB, 1), lambda i: (i, 0)),
            pl.BlockSpec((IB, D), lambda i: (i, 0)),
            pl.BlockSpec((IB, N), lambda i: (i, 0)),
        ],
        out_specs=[
            pl.BlockSpec((IB, 1), lambda i: (i, 0)),
            pl.BlockSpec((IB, D), lambda i: (i, 0)),
            pl.BlockSpec((IB, N), lambda i: (i, 0)),
        ],
        out_shape=[
            jax.ShapeDtypeStruct((N, 1), jnp.int32),
            jax.ShapeDtypeStruct((N, D), jnp.float32),
            jax.ShapeDtypeStruct((N, N), F8),
        ],
    )(v_row, v_col, h, g)

    scat = _sc_scatter_rows(hv, rank_i)

    rank_row = rank_i.reshape(1, N)

    a = pl.pallas_call(
        _build_a_kernel,
        grid=(N // AB,),
        in_specs=[
            pl.BlockSpec((1, N), lambda i: (0, 0)),
            pl.BlockSpec((N, AB), lambda i: (0, i)),
        ],
        out_specs=pl.BlockSpec((K, AB), lambda i: (0, i)),
        out_shape=jax.ShapeDtypeStruct((K, N), F8),
        scratch_shapes=[pltpu.VMEM((K, N), F8)],
    )(rank_row, gf8)

    g_out, idx_row = pl.pallas_call(
        _twohop_kernel,
        grid=(NMB + NJB,),
        in_specs=[
            pl.BlockSpec((K, N), lambda s: (0, 0)),
            pl.BlockSpec((N, MB), lambda s: (0, jnp.minimum(s, NMB - 1))),
            pl.BlockSpec((1, N), lambda s: (0, 0)),
            pl.BlockSpec((N, 1), lambda s: (0, 0)),
        ],
        out_specs=[
            pl.BlockSpec((K, JB), lambda s: (0, jnp.maximum(s - NMB, 0))),
            pl.BlockSpec((1, JB), lambda s: (0, jnp.maximum(s - NMB, 0))),
        ],
        out_shape=[
            jax.ShapeDtypeStruct((K, K), jnp.float32),
            jax.ShapeDtypeStruct((1, K), jnp.int32),
        ],
        scratch_shapes=[pltpu.VMEM((K, N), F8),
                        pltpu.VMEM((1, K), jnp.float32)],
    )(a, gf8, rank_row, rank_i)

    return (g_out, scat[:K], idx_row.reshape(K))


# speedup vs baseline: 1.0346x; 1.0121x over previous
import jax
import jax.numpy as jnp
from jax.experimental import pallas as pl
from jax.experimental.pallas import tpu as pltpu
from jax.experimental.pallas import tpu_sc as plsc

N = 4096
D = 128
K = 2048

IB = 512
AB = 512
MB = 512
JB = 512
NMB = N // MB
NJB = K // JB
WIN = 128

F8 = jnp.float8_e4m3fn


def _rank_kernel(v_row_ref, v_col_ref, h_ref, g_ref, rank_i_ref, hv_ref,
                 gf8_ref):
    pid = pl.program_id(0)
    v_row = v_row_ref[...]
    v_col = v_col_ref[...]
    gt = v_row > v_col
    eq = v_row == v_col
    jj = jax.lax.broadcasted_iota(jnp.int32, (IB, N), 1)
    ii = jax.lax.broadcasted_iota(jnp.int32, (IB, N), 0) + pid * IB
    cnt = gt.astype(jnp.float32) + (eq & (jj < ii)).astype(jnp.float32)
    rank = jnp.sum(cnt, axis=1, keepdims=True)
    rank_i_ref[...] = rank.astype(jnp.int32)
    hv_ref[...] = h_ref[...] * v_col
    gf8_ref[...] = (g_ref[...] != 0).astype(F8)


def _build_a_kernel(rank_row_ref, gf8_ref, a_ref, m_scratch):
    pid = pl.program_id(0)

    @pl.when(pid == 0)
    def _():
        rank_row = rank_row_ref[...]
        for rc in range(K // IB):
            rr = jax.lax.broadcasted_iota(jnp.int32, (IB, N), 0) + (rc * IB)
            m_scratch[rc * IB:(rc + 1) * IB, :] = (rank_row == rr).astype(F8)

    a_ref[...] = jnp.dot(m_scratch[...], gf8_ref[...],
                         preferred_element_type=jnp.float32).astype(F8)


def _twohop_kernel(a_ref, gf8_ref, rank_row_ref, rank_col_ref, out_ref,
                   idx_ref, db_s, deg_s):
    s = pl.program_id(0)

    @pl.when(s < NMB)
    def _():
        d = jnp.dot(a_ref[...], gf8_ref[...],
                    preferred_element_type=jnp.float32)
        db_s[:, pl.ds(s * MB, MB)] = (d != 0.0).astype(F8)

    @pl.when(s == NMB - 1)
    def _():
        sel_row = (rank_row_ref[...] < K).astype(F8)
        deg_s[...] = jax.lax.dot_general(
            sel_row, db_s[...], (((1,), (1,)), ((), ())),
            preferred_element_type=jnp.float32)

    @pl.when(s >= NMB)
    def _():
        jc = s - NMB
        rank_col = rank_col_ref[...]
        jj = jax.lax.broadcasted_iota(jnp.int32, (N, JB), 1) + jc * JB
        pb = rank_col == jj
        c = jnp.dot(db_s[...], pb.astype(F8), preferred_element_type=jnp.float32)
        out_ref[...] = c / deg_s[0:1, pl.ds(jc * JB, JB)]
        mi = jax.lax.broadcasted_iota(jnp.int32, (N, JB), 0)
        idx_ref[...] = jnp.sum(jnp.where(pb, mi, 0), axis=0, keepdims=True)


def _sc_scatter_rows(hv, rank_i32):
    rank2 = rank_i32.reshape(1, N)
    mesh = plsc.VectorSubcoreMesh(core_axis_name="c", subcore_axis_name="s")

    @pl.kernel(out_type=jax.ShapeDtypeStruct((N, D), jnp.float32), mesh=mesh)
    def k(hv_hbm, r_hbm, o_hbm):
        def body(hv_vmem, r_vmem):
            pltpu.sync_copy(hv_vmem, o_hbm.at[r_vmem.at[0]])

        pltpu.emit_pipeline(
            body,
            grid=(N // WIN,),
            in_specs=[pl.BlockSpec((WIN, D), lambda i: (i, 0)),
                      pl.BlockSpec((1, WIN), lambda i: (0, i))],
            out_specs=[],
            core_axis_name=("c", "s"),
            dimension_semantics=(pltpu.PARALLEL,),
        )(hv_hbm, r_hbm)

    return k(hv, rank2)


def kernel(g, h, W, b):
    weights = (h @ W + b).squeeze(-1)
    v = jax.nn.sigmoid(weights)
    v_row = v.reshape(1, N)
    v_col = v.reshape(N, 1)

    rank_i, hv, gf8 = pl.pallas_call(
        _rank_kernel,
        grid=(N // IB,),
        in_specs=[
            pl.BlockSpec((1, N), lambda i: (0, 0)),
            pl.BlockSpec((IB, 1), lambda i: (i, 0)),
            pl.BlockSpec((IB, D), lambda i: (i, 0)),
            pl.BlockSpec((IB, N), lambda i: (i, 0)),
        ],
        out_specs=[
            pl.BlockSpec((IB, 1), lambda i: (i, 0)),
            pl.BlockSpec((IB, D), lambda i: (i, 0)),
            pl.BlockSpec((IB, N), lambda i: (i, 0)),
        ],
        out_shape=[
            jax.ShapeDtypeStruct((N, 1), jnp.int32),
            jax.ShapeDtypeStruct((N, D), jnp.float32),
            jax.ShapeDtypeStruct((N, N), F8),
        ],
    )(v_row, v_col, h, g)

    scat = _sc_scatter_rows(hv, rank_i)

    rank_row = rank_i.reshape(1, N)

    a = pl.pallas_call(
        _build_a_kernel,
        grid=(N // AB,),
        in_specs=[
            pl.BlockSpec((1, N), lambda i: (0, 0)),
            pl.BlockSpec((N, AB), lambda i: (0, i)),
        ],
        out_specs=pl.BlockSpec((K, AB), lambda i: (0, i)),
        out_shape=jax.ShapeDtypeStruct((K, N), F8),
        scratch_shapes=[pltpu.VMEM((K, N), F8)],
    )(rank_row, gf8)

    g_out, idx_row = pl.pallas_call(
        _twohop_kernel,
        grid=(NMB + NJB,),
        in_specs=[
            pl.BlockSpec((K, N), lambda s: (0, 0)),
            pl.BlockSpec((N, MB), lambda s: (0, jnp.minimum(s, NMB - 1))),
            pl.BlockSpec((1, N), lambda s: (0, 0)),
            pl.BlockSpec((N, 1), lambda s: (0, 0)),
        ],
        out_specs=[
            pl.BlockSpec((K, JB), lambda s: (0, jnp.maximum(s - NMB, 0))),
            pl.BlockSpec((1, JB), lambda s: (0, jnp.maximum(s - NMB, 0))),
        ],
        out_shape=[
            jax.ShapeDtypeStruct((K, K), jnp.float32),
            jax.ShapeDtypeStruct((1, K), jnp.int32),
        ],
        scratch_shapes=[pltpu.VMEM((K, N), F8),
                        pltpu.VMEM((1, K), jnp.float32)],
    )(a, gf8, rank_row, rank_i)

    return (g_out, scat[:K], idx_row.reshape(K))
